# Initial kernel scaffold; baseline (speedup 1.0000x reference)
import jax
import jax.numpy as jnp
from jax.experimental import pallas as pl


def kernel(points_1, features_1, points_2, features_2, W1, gamma1, beta1, W2, gamma2, beta2):
    raise NotImplementedError("write your pallas kernel here")


# speedup vs baseline: 12.4412x; 12.4412x over previous
import functools

import jax
import jax.numpy as jnp
from jax import lax
from jax.experimental import pallas as pl
from jax.experimental.pallas import tpu as pltpu
from jax.experimental.pallas import tpu_sc as plsc

B, N1, N2, C1, C2 = 8, 4096, 1024, 128, 256
CO1, CO2 = 256, 256
IN_CH = C1 + C2
TN = 512
Q = B * N1
NW = 32
QPW = Q // NW
CH = 32
NCH = QPW // CH


def _knn_body(p1_ref, p2_ref, idx_ref, w_ref):
    b = pl.program_id(0)
    p1 = p1_ref[0]
    p2 = p2_ref[0]
    dot = lax.dot_general(p1, p2, (((1,), (1,)), ((), ())),
                          preferred_element_type=jnp.float32)
    p1s = jnp.sum(p1 * p1, axis=1, keepdims=True)
    p2s = jnp.sum(p2 * p2, axis=1)[None, :]
    d2 = p1s + p2s - 2.0 * dot
    iota = lax.broadcasted_iota(jnp.int32, (TN, N2), 1)
    cur = d2
    vals, idxs = [], []
    for _ in range(3):
        mv = jnp.min(cur, axis=1, keepdims=True)
        mi = jnp.min(jnp.where(cur == mv, iota, N2), axis=1, keepdims=True)
        vals.append(mv)
        idxs.append(mi)
        cur = jnp.where(iota == mi, jnp.float32(jnp.inf), cur)
    dist = jnp.concatenate(vals, axis=1)
    ind = jnp.concatenate(idxs, axis=1)
    rec = 1.0 / (dist + 1e-8)
    w_ref[0] = rec / jnp.sum(rec, axis=1, keepdims=True)
    idx_ref[0] = ind + b * N2


def _knn(points_1, points_2):
    return pl.pallas_call(
        _knn_body,
        grid=(B, N1 // TN),
        in_specs=[
            pl.BlockSpec((1, TN, 3), lambda b, t: (b, t, 0)),
            pl.BlockSpec((1, N2, 3), lambda b, t: (b, 0, 0)),
        ],
        out_specs=[
            pl.BlockSpec((1, TN, 3), lambda b, t: (b, t, 0)),
            pl.BlockSpec((1, TN, 3), lambda b, t: (b, t, 0)),
        ],
        out_shape=[
            jax.ShapeDtypeStruct((B, N1, 3), jnp.int32),
            jax.ShapeDtypeStruct((B, N1, 3), jnp.float32),
        ],
    )(points_1, points_2)


@functools.lru_cache(maxsize=1)
def _make_gather_combine():
    mesh = plsc.VectorSubcoreMesh(core_axis_name="c", subcore_axis_name="s")

    @functools.partial(
        pl.kernel,
        mesh=mesh,
        out_type=jax.ShapeDtypeStruct((Q, C2), jnp.float32),
        scratch_types=[
            pltpu.VMEM((3 * CH,), jnp.int32),
            pltpu.VMEM((3 * CH + 16,), jnp.float32),
            pltpu.VMEM((3 * CH, C2), jnp.float32),
            pltpu.VMEM((CH, C2), jnp.float32),
            pltpu.SemaphoreType.DMA,
        ],
    )
    def gather_combine(f2_hbm, idx_hbm, w_hbm, out_hbm,
                       idx_v, w_v, rows_v, out_v, sem):
        wid = lax.axis_index("s") * 2 + lax.axis_index("c")
        qbase = wid * QPW

        def chunk(ci, carry):
            q0 = qbase + ci * CH
            pltpu.sync_copy(idx_hbm.at[pl.ds(3 * q0, 3 * CH)], idx_v)
            pltpu.sync_copy(w_hbm.at[pl.ds(3 * q0, 3 * CH)],
                            w_v.at[pl.ds(0, 3 * CH)])
            pltpu.async_copy(f2_hbm.at[idx_v], rows_v, sem).wait()

            def per_q(qi, c2):
                wvec = w_v[pl.ds(3 * qi, 16)]
                w0 = wvec[0]
                w1 = wvec[1]
                w2 = wvec[2]
                for j in range(C2 // 16):
                    sl = pl.ds(j * 16, 16)
                    out_v[qi, sl] = (w0 * rows_v[3 * qi, sl]
                                     + w1 * rows_v[3 * qi + 1, sl]
                                     + w2 * rows_v[3 * qi + 2, sl])
                return c2

            lax.fori_loop(0, CH, per_q, 0)
            pltpu.sync_copy(out_v, out_hbm.at[pl.ds(q0, CH)])
            return carry

        lax.fori_loop(0, NCH, chunk, 0)

    return gather_combine


def _sc_gather(f2flat, idxf, wf):
    return _make_gather_combine()(f2flat, idxf, wf)


def _mlp1_body(recv_ref, f1_ref, w1_ref, h_ref, st_ref):
    x = jnp.concatenate([recv_ref[0], f1_ref[0]], axis=1)
    h = lax.dot_general(x, w1_ref[...], (((1,), (1,)), ((), ())),
                        preferred_element_type=jnp.float32)

    @pl.when((pl.program_id(0) == 0) & (pl.program_id(1) == 0))
    def _init():
        st_ref[...] = jnp.zeros_like(st_ref)

    h_ref[0] = h
    st_ref[0:1, :] += jnp.sum(h, axis=0, keepdims=True)
    st_ref[1:2, :] += jnp.sum(h * h, axis=0, keepdims=True)


def _mlp2_body(h1_ref, st1_ref, g1_ref, b1_ref, w2_ref, h2_ref, st2_ref):
    n = jnp.float32(B * N1)
    mean = st1_ref[0:1, :] / n
    var = st1_ref[1:2, :] / n - mean * mean
    scale = g1_ref[...] * lax.rsqrt(var + 1e-5)
    shift = b1_ref[...] - mean * scale
    x = jnp.maximum(h1_ref[0] * scale + shift, 0.0)
    h2 = lax.dot_general(x, w2_ref[...], (((1,), (1,)), ((), ())),
                         preferred_element_type=jnp.float32)

    @pl.when((pl.program_id(0) == 0) & (pl.program_id(1) == 0))
    def _init():
        st2_ref[...] = jnp.zeros_like(st2_ref)

    h2_ref[0] = h2
    st2_ref[0:1, :] += jnp.sum(h2, axis=0, keepdims=True)
    st2_ref[1:2, :] += jnp.sum(h2 * h2, axis=0, keepdims=True)


def _bn_relu_body(h_ref, st_ref, g_ref, b_ref, o_ref):
    n = jnp.float32(B * N1)
    mean = st_ref[0:1, :] / n
    var = st_ref[1:2, :] / n - mean * mean
    scale = g_ref[...] * lax.rsqrt(var + 1e-5)
    shift = b_ref[...] - mean * scale
    o_ref[0] = jnp.maximum(h_ref[0] * scale + shift, 0.0)


def _row_spec():
    return pl.BlockSpec((1, TN, CO1), lambda b, t: (b, t, 0))


def _full_spec(shape):
    return pl.BlockSpec(shape, lambda b, t: tuple(0 for _ in shape))


def _mlp1_call(recv, features_1, W1):
    return pl.pallas_call(
        _mlp1_body,
        grid=(B, N1 // TN),
        in_specs=[
            pl.BlockSpec((1, TN, C2), lambda b, t: (b, t, 0)),
            pl.BlockSpec((1, TN, C1), lambda b, t: (b, t, 0)),
            _full_spec((CO1, IN_CH)),
        ],
        out_specs=[_row_spec(), _full_spec((2, CO1))],
        out_shape=[
            jax.ShapeDtypeStruct((B, N1, CO1), jnp.float32),
            jax.ShapeDtypeStruct((2, CO1), jnp.float32),
        ],
    )(recv, features_1, W1)


def _mlp2_call(h1, st1, gamma1, beta1, W2):
    return pl.pallas_call(
        _mlp2_body,
        grid=(B, N1 // TN),
        in_specs=[
            _row_spec(),
            _full_spec((2, CO1)),
            _full_spec((1, CO1)),
            _full_spec((1, CO1)),
            _full_spec((CO2, CO1)),
        ],
        out_specs=[_row_spec(), _full_spec((2, CO2))],
        out_shape=[
            jax.ShapeDtypeStruct((B, N1, CO2), jnp.float32),
            jax.ShapeDtypeStruct((2, CO2), jnp.float32),
        ],
    )(h1, st1, gamma1.reshape(1, CO1), beta1.reshape(1, CO1), W2)


def _bn3_call(h2, st2, gamma2, beta2):
    return pl.pallas_call(
        _bn_relu_body,
        grid=(B, N1 // TN),
        in_specs=[
            _row_spec(),
            _full_spec((2, CO2)),
            _full_spec((1, CO2)),
            _full_spec((1, CO2)),
        ],
        out_specs=_row_spec(),
        out_shape=jax.ShapeDtypeStruct((B, N1, CO2), jnp.float32),
    )(h2, st2, gamma2.reshape(1, CO2), beta2.reshape(1, CO2))


def kernel(points_1, features_1, points_2, features_2,
           W1, gamma1, beta1, W2, gamma2, beta2):
    idx, w = _knn(points_1, points_2)
    recv = _sc_gather(features_2.reshape(B * N2, C2),
                      idx.reshape(Q * 3), w.reshape(Q * 3))
    h1, st1 = _mlp1_call(recv.reshape(B, N1, C2), features_1, W1)
    h2, st2 = _mlp2_call(h1, st1, gamma1, beta1, W2)
    return _bn3_call(h2, st2, gamma2, beta2)


# speedup vs baseline: 14.7056x; 1.1820x over previous
import functools

import jax
import jax.numpy as jnp
from jax import lax
from jax.experimental import pallas as pl
from jax.experimental.pallas import tpu as pltpu
from jax.experimental.pallas import tpu_sc as plsc

B, N1, N2, C1, C2 = 8, 4096, 1024, 128, 256
CO1, CO2 = 256, 256
IN_CH = C1 + C2
TN = 512
Q = B * N1
NW = 32
QPW = Q // NW
CH = 32
NCH = QPW // CH


def _knn_body(p1_ref, p2_ref, idx_ref, w_ref):
    b = pl.program_id(0)
    p1 = p1_ref[0]
    p2 = p2_ref[0]
    dot = lax.dot_general(p1, p2, (((1,), (1,)), ((), ())),
                          preferred_element_type=jnp.float32)
    p1s = jnp.sum(p1 * p1, axis=1, keepdims=True)
    p2s = jnp.sum(p2 * p2, axis=1)[None, :]
    d2 = p1s + p2s - 2.0 * dot
    iota = lax.broadcasted_iota(jnp.int32, (TN, N2), 1)
    cur = d2
    vals, idxs = [], []
    for _ in range(3):
        mv = jnp.min(cur, axis=1, keepdims=True)
        mi = jnp.min(jnp.where(cur == mv, iota, N2), axis=1, keepdims=True)
        vals.append(mv)
        idxs.append(mi)
        cur = jnp.where(iota == mi, jnp.float32(jnp.inf), cur)
    dist = jnp.concatenate(vals, axis=1)
    ind = jnp.concatenate(idxs, axis=1)
    rec = 1.0 / (dist + 1e-8)
    w_ref[0] = rec / jnp.sum(rec, axis=1, keepdims=True)
    idx_ref[0] = ind + b * N2


def _knn(points_1, points_2):
    return pl.pallas_call(
        _knn_body,
        grid=(B, N1 // TN),
        in_specs=[
            pl.BlockSpec((1, TN, 3), lambda b, t: (b, t, 0)),
            pl.BlockSpec((1, N2, 3), lambda b, t: (b, 0, 0)),
        ],
        out_specs=[
            pl.BlockSpec((1, TN, 3), lambda b, t: (b, t, 0)),
            pl.BlockSpec((1, TN, 3), lambda b, t: (b, t, 0)),
        ],
        out_shape=[
            jax.ShapeDtypeStruct((B, N1, 3), jnp.int32),
            jax.ShapeDtypeStruct((B, N1, 3), jnp.float32),
        ],
    )(points_1, points_2)


@functools.lru_cache(maxsize=1)
def _make_gather_combine():
    mesh = plsc.VectorSubcoreMesh(core_axis_name="c", subcore_axis_name="s")

    @functools.partial(
        pl.kernel,
        mesh=mesh,
        out_type=jax.ShapeDtypeStruct((Q, C2), jnp.float32),
        scratch_types=[
            pltpu.VMEM((3 * QPW,), jnp.int32),
            pltpu.VMEM((3 * QPW + 16,), jnp.float32),
            pltpu.VMEM((2, 3 * CH, C2), jnp.float32),
            pltpu.VMEM((2, CH, C2), jnp.float32),
            pltpu.SemaphoreType.DMA,
            pltpu.SemaphoreType.DMA,
        ],
    )
    def gather_combine(f2_hbm, idx_hbm, w_hbm, out_hbm,
                       idx_v, w_v, rows_v, out_v, gsem, osem):
        wid = lax.axis_index("s") * 2 + lax.axis_index("c")
        qbase = wid * QPW
        pltpu.sync_copy(idx_hbm.at[pl.ds(3 * qbase, 3 * QPW)], idx_v)
        pltpu.sync_copy(w_hbm.at[pl.ds(3 * qbase, 3 * QPW)],
                        w_v.at[pl.ds(0, 3 * QPW)])

        def start_gather(ci, slot):
            pltpu.async_copy(f2_hbm.at[idx_v.at[pl.ds(3 * CH * ci, 3 * CH)]],
                             rows_v.at[slot], gsem)

        def combine(ci, slot):
            def per_q(qi, c):
                wvec = w_v[pl.ds(3 * CH * ci + 3 * qi, 16)]
                w0 = wvec[0]
                w1 = wvec[1]
                w2 = wvec[2]
                for j in range(C2 // 16):
                    sl = pl.ds(j * 16, 16)
                    out_v[slot, qi, sl] = (w0 * rows_v[slot, 3 * qi, sl]
                                           + w1 * rows_v[slot, 3 * qi + 1, sl]
                                           + w2 * rows_v[slot, 3 * qi + 2, sl])
                return c

            lax.fori_loop(0, CH, per_q, 0)

        def wait_gather(slot):
            pltpu.make_async_copy(f2_hbm.at[pl.ds(0, 3 * CH)],
                                  rows_v.at[slot], gsem).wait()

        def wait_out(slot):
            pltpu.make_async_copy(out_v.at[slot],
                                  out_hbm.at[pl.ds(qbase, CH)], osem).wait()

        start_gather(0, 0)

        def step(i2, carry):
            for b in range(2):
                ci = 2 * i2 + b
                slot = b
                nxt = ci + 1

                @pl.when(nxt < NCH)
                def _pre():
                    start_gather(nxt, 1 - slot)

                wait_gather(slot)

                @pl.when(ci >= 2)
                def _drain():
                    wait_out(slot)

                combine(ci, slot)
                pltpu.async_copy(out_v.at[slot],
                                 out_hbm.at[pl.ds(qbase + ci * CH, CH)], osem)
            return carry

        lax.fori_loop(0, NCH // 2, step, 0)
        wait_out(0)
        wait_out(1)

    return gather_combine


def _sc_gather(f2flat, idxf, wf):
    return _make_gather_combine()(f2flat, idxf, wf)


def _mlp1_body(recv_ref, f1_ref, w1_ref, h_ref, st_ref):
    x = jnp.concatenate([recv_ref[0], f1_ref[0]], axis=1)
    h = lax.dot_general(x, w1_ref[...], (((1,), (1,)), ((), ())),
                        preferred_element_type=jnp.float32)

    @pl.when((pl.program_id(0) == 0) & (pl.program_id(1) == 0))
    def _init():
        st_ref[...] = jnp.zeros_like(st_ref)

    h_ref[0] = h
    st_ref[0:1, :] += jnp.sum(h, axis=0, keepdims=True)
    st_ref[1:2, :] += jnp.sum(h * h, axis=0, keepdims=True)


def _mlp2_body(h1_ref, st1_ref, g1_ref, b1_ref, w2_ref, h2_ref, st2_ref):
    n = jnp.float32(B * N1)
    mean = st1_ref[0:1, :] / n
    var = st1_ref[1:2, :] / n - mean * mean
    scale = g1_ref[...] * lax.rsqrt(var + 1e-5)
    shift = b1_ref[...] - mean * scale
    x = jnp.maximum(h1_ref[0] * scale + shift, 0.0)
    h2 = lax.dot_general(x, w2_ref[...], (((1,), (1,)), ((), ())),
                         preferred_element_type=jnp.float32)

    @pl.when((pl.program_id(0) == 0) & (pl.program_id(1) == 0))
    def _init():
        st2_ref[...] = jnp.zeros_like(st2_ref)

    h2_ref[0] = h2
    st2_ref[0:1, :] += jnp.sum(h2, axis=0, keepdims=True)
    st2_ref[1:2, :] += jnp.sum(h2 * h2, axis=0, keepdims=True)


def _bn_relu_body(h_ref, st_ref, g_ref, b_ref, o_ref):
    n = jnp.float32(B * N1)
    mean = st_ref[0:1, :] / n
    var = st_ref[1:2, :] / n - mean * mean
    scale = g_ref[...] * lax.rsqrt(var + 1e-5)
    shift = b_ref[...] - mean * scale
    o_ref[0] = jnp.maximum(h_ref[0] * scale + shift, 0.0)


def _row_spec():
    return pl.BlockSpec((1, TN, CO1), lambda b, t: (b, t, 0))


def _full_spec(shape):
    return pl.BlockSpec(shape, lambda b, t: tuple(0 for _ in shape))


def _mlp1_call(recv, features_1, W1):
    return pl.pallas_call(
        _mlp1_body,
        grid=(B, N1 // TN),
        in_specs=[
            pl.BlockSpec((1, TN, C2), lambda b, t: (b, t, 0)),
            pl.BlockSpec((1, TN, C1), lambda b, t: (b, t, 0)),
            _full_spec((CO1, IN_CH)),
        ],
        out_specs=[_row_spec(), _full_spec((2, CO1))],
        out_shape=[
            jax.ShapeDtypeStruct((B, N1, CO1), jnp.float32),
            jax.ShapeDtypeStruct((2, CO1), jnp.float32),
        ],
    )(recv, features_1, W1)


def _mlp2_call(h1, st1, gamma1, beta1, W2):
    return pl.pallas_call(
        _mlp2_body,
        grid=(B, N1 // TN),
        in_specs=[
            _row_spec(),
            _full_spec((2, CO1)),
            _full_spec((1, CO1)),
            _full_spec((1, CO1)),
            _full_spec((CO2, CO1)),
        ],
        out_specs=[_row_spec(), _full_spec((2, CO2))],
        out_shape=[
            jax.ShapeDtypeStruct((B, N1, CO2), jnp.float32),
            jax.ShapeDtypeStruct((2, CO2), jnp.float32),
        ],
    )(h1, st1, gamma1.reshape(1, CO1), beta1.reshape(1, CO1), W2)


def _bn3_call(h2, st2, gamma2, beta2):
    return pl.pallas_call(
        _bn_relu_body,
        grid=(B, N1 // TN),
        in_specs=[
            _row_spec(),
            _full_spec((2, CO2)),
            _full_spec((1, CO2)),
            _full_spec((1, CO2)),
        ],
        out_specs=_row_spec(),
        out_shape=jax.ShapeDtypeStruct((B, N1, CO2), jnp.float32),
    )(h2, st2, gamma2.reshape(1, CO2), beta2.reshape(1, CO2))


def kernel(points_1, features_1, points_2, features_2,
           W1, gamma1, beta1, W2, gamma2, beta2):
    idx, w = _knn(points_1, points_2)
    recv = _sc_gather(features_2.reshape(B * N2, C2),
                      idx.reshape(Q * 3), w.reshape(Q * 3))
    h1, st1 = _mlp1_call(recv.reshape(B, N1, C2), features_1, W1)
    h2, st2 = _mlp2_call(h1, st1, gamma1, beta1, W2)
    return _bn3_call(h2, st2, gamma2, beta2)


# speedup vs baseline: 16.5153x; 1.1231x over previous
import functools

import jax
import jax.numpy as jnp
from jax import lax
from jax.experimental import pallas as pl
from jax.experimental.pallas import tpu as pltpu
from jax.experimental.pallas import tpu_sc as plsc

B, N1, N2, C1, C2 = 8, 4096, 1024, 128, 256
CO1, CO2 = 256, 256
IN_CH = C1 + C2
TN = 1024
Q = B * N1
NW = 32
QPW = Q // NW
CH = 32
NCH = QPW // CH


def _knn_body(p1_ref, p2_ref, idx_ref, w_ref):
    b = pl.program_id(0)
    p1 = p1_ref[0]
    p2 = p2_ref[0]
    dot = lax.dot_general(p1, p2, (((1,), (1,)), ((), ())),
                          preferred_element_type=jnp.float32)
    p1s = jnp.sum(p1 * p1, axis=1, keepdims=True)
    p2s = jnp.sum(p2 * p2, axis=1)[None, :]
    d2 = p1s + p2s - 2.0 * dot
    iota = lax.broadcasted_iota(jnp.int32, (TN, N2), 1)
    cur = d2
    vals, idxs = [], []
    for _ in range(3):
        mv = jnp.min(cur, axis=1, keepdims=True)
        mi = jnp.min(jnp.where(cur == mv, iota, N2), axis=1, keepdims=True)
        vals.append(mv)
        idxs.append(mi)
        cur = jnp.where(iota == mi, jnp.float32(jnp.inf), cur)
    dist = jnp.concatenate(vals, axis=1)
    ind = jnp.concatenate(idxs, axis=1)
    rec = 1.0 / (dist + 1e-8)
    w_ref[0] = rec / jnp.sum(rec, axis=1, keepdims=True)
    idx_ref[0] = ind + b * N2


def _knn(points_1, points_2):
    return pl.pallas_call(
        _knn_body,
        grid=(B, N1 // TN),
        in_specs=[
            pl.BlockSpec((1, TN, 3), lambda b, t: (b, t, 0)),
            pl.BlockSpec((1, N2, 3), lambda b, t: (b, 0, 0)),
        ],
        out_specs=[
            pl.BlockSpec((1, TN, 3), lambda b, t: (b, t, 0)),
            pl.BlockSpec((1, TN, 3), lambda b, t: (b, t, 0)),
        ],
        out_shape=[
            jax.ShapeDtypeStruct((B, N1, 3), jnp.int32),
            jax.ShapeDtypeStruct((B, N1, 3), jnp.float32),
        ],
    )(points_1, points_2)


@functools.lru_cache(maxsize=1)
def _make_gather_combine():
    mesh = plsc.VectorSubcoreMesh(core_axis_name="c", subcore_axis_name="s")

    @functools.partial(
        pl.kernel,
        mesh=mesh,
        out_type=jax.ShapeDtypeStruct((Q, C2), jnp.float32),
        scratch_types=[
            pltpu.VMEM((3 * QPW,), jnp.int32),
            pltpu.VMEM((3 * QPW + 16,), jnp.float32),
            pltpu.VMEM((2, 3 * CH, C2), jnp.float32),
            pltpu.VMEM((2, CH, C2), jnp.float32),
            pltpu.SemaphoreType.DMA,
            pltpu.SemaphoreType.DMA,
        ],
    )
    def gather_combine(f2_hbm, idx_hbm, w_hbm, out_hbm,
                       idx_v, w_v, rows_v, out_v, gsem, osem):
        wid = lax.axis_index("s") * 2 + lax.axis_index("c")
        qbase = wid * QPW
        pltpu.sync_copy(idx_hbm.at[pl.ds(3 * qbase, 3 * QPW)], idx_v)
        pltpu.sync_copy(w_hbm.at[pl.ds(3 * qbase, 3 * QPW)],
                        w_v.at[pl.ds(0, 3 * QPW)])

        def start_gather(ci, slot):
            pltpu.async_copy(f2_hbm.at[idx_v.at[pl.ds(3 * CH * ci, 3 * CH)]],
                             rows_v.at[slot], gsem)

        def combine(ci, slot):
            def per_q(qi, c):
                wvec = w_v[pl.ds(3 * CH * ci + 3 * qi, 16)]
                w0 = wvec[0]
                w1 = wvec[1]
                w2 = wvec[2]
                for j in range(C2 // 16):
                    sl = pl.ds(j * 16, 16)
                    out_v[slot, qi, sl] = (w0 * rows_v[slot, 3 * qi, sl]
                                           + w1 * rows_v[slot, 3 * qi + 1, sl]
                                           + w2 * rows_v[slot, 3 * qi + 2, sl])
                return c

            lax.fori_loop(0, CH, per_q, 0)

        def wait_gather(slot):
            pltpu.make_async_copy(f2_hbm.at[pl.ds(0, 3 * CH)],
                                  rows_v.at[slot], gsem).wait()

        def wait_out(slot):
            pltpu.make_async_copy(out_v.at[slot],
                                  out_hbm.at[pl.ds(qbase, CH)], osem).wait()

        start_gather(0, 0)

        def step(i2, carry):
            for b in range(2):
                ci = 2 * i2 + b
                slot = b
                nxt = ci + 1

                @pl.when(nxt < NCH)
                def _pre():
                    start_gather(nxt, 1 - slot)

                wait_gather(slot)

                @pl.when(ci >= 2)
                def _drain():
                    wait_out(slot)

                combine(ci, slot)
                pltpu.async_copy(out_v.at[slot],
                                 out_hbm.at[pl.ds(qbase + ci * CH, CH)], osem)
            return carry

        lax.fori_loop(0, NCH // 2, step, 0)
        wait_out(0)
        wait_out(1)

    return gather_combine


def _sc_gather(f2flat, idxf, wf):
    return _make_gather_combine()(f2flat, idxf, wf)


def _mlp1_body(recv_ref, f1_ref, w1_ref, h_ref, st_ref):
    x = jnp.concatenate([recv_ref[0], f1_ref[0]], axis=1)
    h = lax.dot_general(x, w1_ref[...], (((1,), (1,)), ((), ())),
                        preferred_element_type=jnp.float32)

    @pl.when((pl.program_id(0) == 0) & (pl.program_id(1) == 0))
    def _init():
        st_ref[...] = jnp.zeros_like(st_ref)

    h_ref[0] = h
    st_ref[0:1, :] += jnp.sum(h, axis=0, keepdims=True)
    st_ref[1:2, :] += jnp.sum(h * h, axis=0, keepdims=True)


def _mlp2_body(h1_ref, st1_ref, g1_ref, b1_ref, w2_ref, h2_ref, st2_ref):
    n = jnp.float32(B * N1)
    mean = st1_ref[0:1, :] / n
    var = st1_ref[1:2, :] / n - mean * mean
    scale = g1_ref[...] * lax.rsqrt(var + 1e-5)
    shift = b1_ref[...] - mean * scale
    x = jnp.maximum(h1_ref[0] * scale + shift, 0.0)
    h2 = lax.dot_general(x, w2_ref[...], (((1,), (1,)), ((), ())),
                         preferred_element_type=jnp.float32)

    @pl.when((pl.program_id(0) == 0) & (pl.program_id(1) == 0))
    def _init():
        st2_ref[...] = jnp.zeros_like(st2_ref)

    h2_ref[0] = h2
    st2_ref[0:1, :] += jnp.sum(h2, axis=0, keepdims=True)
    st2_ref[1:2, :] += jnp.sum(h2 * h2, axis=0, keepdims=True)


def _bn_relu_body(h_ref, st_ref, g_ref, b_ref, o_ref):
    n = jnp.float32(B * N1)
    mean = st_ref[0:1, :] / n
    var = st_ref[1:2, :] / n - mean * mean
    scale = g_ref[...] * lax.rsqrt(var + 1e-5)
    shift = b_ref[...] - mean * scale
    o_ref[0] = jnp.maximum(h_ref[0] * scale + shift, 0.0)


def _row_spec():
    return pl.BlockSpec((1, TN, CO1), lambda b, t: (b, t, 0))


def _full_spec(shape):
    return pl.BlockSpec(shape, lambda b, t: tuple(0 for _ in shape))


def _mlp1_call(recv, features_1, W1):
    return pl.pallas_call(
        _mlp1_body,
        grid=(B, N1 // TN),
        in_specs=[
            pl.BlockSpec((1, TN, C2), lambda b, t: (b, t, 0)),
            pl.BlockSpec((1, TN, C1), lambda b, t: (b, t, 0)),
            _full_spec((CO1, IN_CH)),
        ],
        out_specs=[_row_spec(), _full_spec((2, CO1))],
        out_shape=[
            jax.ShapeDtypeStruct((B, N1, CO1), jnp.float32),
            jax.ShapeDtypeStruct((2, CO1), jnp.float32),
        ],
    )(recv, features_1, W1)


def _mlp2_call(h1, st1, gamma1, beta1, W2):
    return pl.pallas_call(
        _mlp2_body,
        grid=(B, N1 // TN),
        in_specs=[
            _row_spec(),
            _full_spec((2, CO1)),
            _full_spec((1, CO1)),
            _full_spec((1, CO1)),
            _full_spec((CO2, CO1)),
        ],
        out_specs=[_row_spec(), _full_spec((2, CO2))],
        out_shape=[
            jax.ShapeDtypeStruct((B, N1, CO2), jnp.float32),
            jax.ShapeDtypeStruct((2, CO2), jnp.float32),
        ],
    )(h1, st1, gamma1.reshape(1, CO1), beta1.reshape(1, CO1), W2)


def _bn3_call(h2, st2, gamma2, beta2):
    return pl.pallas_call(
        _bn_relu_body,
        grid=(B, N1 // TN),
        in_specs=[
            _row_spec(),
            _full_spec((2, CO2)),
            _full_spec((1, CO2)),
            _full_spec((1, CO2)),
        ],
        out_specs=_row_spec(),
        out_shape=jax.ShapeDtypeStruct((B, N1, CO2), jnp.float32),
    )(h2, st2, gamma2.reshape(1, CO2), beta2.reshape(1, CO2))


def kernel(points_1, features_1, points_2, features_2,
           W1, gamma1, beta1, W2, gamma2, beta2):
    idx, w = _knn(points_1, points_2)
    recv = _sc_gather(features_2.reshape(B * N2, C2),
                      idx.reshape(Q * 3), w.reshape(Q * 3))
    h1, st1 = _mlp1_call(recv.reshape(B, N1, C2), features_1, W1)
    h2, st2 = _mlp2_call(h1, st1, gamma1, beta1, W2)
    return _bn3_call(h2, st2, gamma2, beta2)


# speedup vs baseline: 17.1729x; 1.0398x over previous
import functools

import jax
import jax.numpy as jnp
from jax import lax
from jax.experimental import pallas as pl
from jax.experimental.pallas import tpu as pltpu
from jax.experimental.pallas import tpu_sc as plsc

B, N1, N2, C1, C2 = 8, 4096, 1024, 128, 256
CO1, CO2 = 256, 256
IN_CH = C1 + C2
TN = 1024
Q = B * N1
NW = 32
QPW = Q // NW
CH = 32
NCH = QPW // CH


def _knn_body(p1_ref, p2_ref, idx_ref, w_ref):
    b = pl.program_id(0)
    p1 = p1_ref[0]
    p2 = p2_ref[0]
    dot = lax.dot_general(p1, p2, (((1,), (1,)), ((), ())),
                          preferred_element_type=jnp.float32)
    p1s = jnp.sum(p1 * p1, axis=1, keepdims=True)
    p2s = jnp.sum(p2 * p2, axis=1)[None, :]
    d2 = p1s + p2s - 2.0 * dot
    iota = lax.broadcasted_iota(jnp.int32, (TN, N2), 1)
    cur = d2
    vals, idxs = [], []
    for _ in range(3):
        mv = jnp.min(cur, axis=1, keepdims=True)
        mi = jnp.min(jnp.where(cur == mv, iota, N2), axis=1, keepdims=True)
        vals.append(mv)
        idxs.append(mi)
        cur = jnp.where(iota == mi, jnp.float32(jnp.inf), cur)
    dist = jnp.concatenate(vals, axis=1)
    ind = jnp.concatenate(idxs, axis=1)
    rec = 1.0 / (dist + 1e-8)
    w_ref[0] = rec / jnp.sum(rec, axis=1, keepdims=True)
    idx_ref[0] = ind + b * N2


def _knn(points_1, points_2):
    return pl.pallas_call(
        _knn_body,
        grid=(B, N1 // TN),
        in_specs=[
            pl.BlockSpec((1, TN, 3), lambda b, t: (b, t, 0)),
            pl.BlockSpec((1, N2, 3), lambda b, t: (b, 0, 0)),
        ],
        out_specs=[
            pl.BlockSpec((1, TN, 3), lambda b, t: (b, t, 0)),
            pl.BlockSpec((1, TN, 3), lambda b, t: (b, t, 0)),
        ],
        out_shape=[
            jax.ShapeDtypeStruct((B, N1, 3), jnp.int32),
            jax.ShapeDtypeStruct((B, N1, 3), jnp.float32),
        ],
    )(points_1, points_2)


@functools.lru_cache(maxsize=1)
def _make_gather_combine():
    mesh = plsc.VectorSubcoreMesh(core_axis_name="c", subcore_axis_name="s")

    @functools.partial(
        pl.kernel,
        mesh=mesh,
        out_type=jax.ShapeDtypeStruct((Q, C2), jnp.float32),
        scratch_types=[
            pltpu.VMEM((3 * QPW,), jnp.int32),
            pltpu.VMEM((3 * QPW + 16,), jnp.float32),
            pltpu.VMEM((2, 3 * CH, C2), jnp.float32),
            pltpu.VMEM((2, CH, C2), jnp.float32),
            pltpu.SemaphoreType.DMA,
            pltpu.SemaphoreType.DMA,
        ],
    )
    def gather_combine(f2_hbm, idx_hbm, w_hbm, out_hbm,
                       idx_v, w_v, rows_v, out_v, gsem, osem):
        wid = lax.axis_index("s") * 2 + lax.axis_index("c")
        qbase = wid * QPW
        pltpu.sync_copy(idx_hbm.at[pl.ds(3 * qbase, 3 * QPW)], idx_v)
        pltpu.sync_copy(w_hbm.at[pl.ds(3 * qbase, 3 * QPW)],
                        w_v.at[pl.ds(0, 3 * QPW)])

        def start_gather(ci, slot):
            pltpu.async_copy(f2_hbm.at[idx_v.at[pl.ds(3 * CH * ci, 3 * CH)]],
                             rows_v.at[slot], gsem)

        def combine(ci, slot):
            def per_q(qi, c):
                wvec = w_v[pl.ds(3 * CH * ci + 3 * qi, 16)]
                w0 = wvec[0]
                w1 = wvec[1]
                w2 = wvec[2]
                for j in range(C2 // 16):
                    sl = pl.ds(j * 16, 16)
                    out_v[slot, qi, sl] = (w0 * rows_v[slot, 3 * qi, sl]
                                           + w1 * rows_v[slot, 3 * qi + 1, sl]
                                           + w2 * rows_v[slot, 3 * qi + 2, sl])
                return c

            lax.fori_loop(0, CH, per_q, 0)

        def wait_gather(slot):
            pltpu.make_async_copy(f2_hbm.at[pl.ds(0, 3 * CH)],
                                  rows_v.at[slot], gsem).wait()

        def wait_out(slot):
            pltpu.make_async_copy(out_v.at[slot],
                                  out_hbm.at[pl.ds(qbase, CH)], osem).wait()

        start_gather(0, 0)

        def step(i2, carry):
            for b in range(2):
                ci = 2 * i2 + b
                slot = b
                nxt = ci + 1

                @pl.when(nxt < NCH)
                def _pre():
                    start_gather(nxt, 1 - slot)

                wait_gather(slot)

                @pl.when(ci >= 2)
                def _drain():
                    wait_out(slot)

                combine(ci, slot)
                pltpu.async_copy(out_v.at[slot],
                                 out_hbm.at[pl.ds(qbase + ci * CH, CH)], osem)
            return carry

        lax.fori_loop(0, NCH // 2, step, 0)
        wait_out(0)
        wait_out(1)

    return gather_combine


def _sc_gather(f2flat, idxf, wf):
    return _make_gather_combine()(f2flat, idxf, wf)


def _mlp1_body(recv_ref, f1_ref, w1_ref, h_ref, st_ref):
    x = jnp.concatenate([recv_ref[0], f1_ref[0]], axis=1)
    h = lax.dot_general(x, w1_ref[...], (((1,), (1,)), ((), ())),
                        preferred_element_type=jnp.float32)

    @pl.when((pl.program_id(0) == 0) & (pl.program_id(1) == 0))
    def _init():
        st_ref[...] = jnp.zeros_like(st_ref)

    h_ref[0] = h.astype(jnp.bfloat16)
    st_ref[0:1, :] += jnp.sum(h, axis=0, keepdims=True)
    st_ref[1:2, :] += jnp.sum(h * h, axis=0, keepdims=True)


def _mlp2_body(h1_ref, st1_ref, g1_ref, b1_ref, w2_ref, h2_ref, st2_ref):
    n = jnp.float32(B * N1)
    mean = st1_ref[0:1, :] / n
    var = st1_ref[1:2, :] / n - mean * mean
    scale = g1_ref[...] * lax.rsqrt(var + 1e-5)
    shift = b1_ref[...] - mean * scale
    x = jnp.maximum(h1_ref[0].astype(jnp.float32) * scale + shift, 0.0)
    h2 = lax.dot_general(x, w2_ref[...], (((1,), (1,)), ((), ())),
                         preferred_element_type=jnp.float32)

    @pl.when((pl.program_id(0) == 0) & (pl.program_id(1) == 0))
    def _init():
        st2_ref[...] = jnp.zeros_like(st2_ref)

    h2_ref[0] = h2.astype(jnp.bfloat16)
    st2_ref[0:1, :] += jnp.sum(h2, axis=0, keepdims=True)
    st2_ref[1:2, :] += jnp.sum(h2 * h2, axis=0, keepdims=True)


def _bn_relu_body(h_ref, st_ref, g_ref, b_ref, o_ref):
    n = jnp.float32(B * N1)
    mean = st_ref[0:1, :] / n
    var = st_ref[1:2, :] / n - mean * mean
    scale = g_ref[...] * lax.rsqrt(var + 1e-5)
    shift = b_ref[...] - mean * scale
    o_ref[0] = jnp.maximum(h_ref[0].astype(jnp.float32) * scale + shift, 0.0)


def _row_spec():
    return pl.BlockSpec((1, TN, CO1), lambda b, t: (b, t, 0))


def _full_spec(shape):
    return pl.BlockSpec(shape, lambda b, t: tuple(0 for _ in shape))


def _mlp1_call(recv, features_1, W1):
    return pl.pallas_call(
        _mlp1_body,
        grid=(B, N1 // TN),
        in_specs=[
            pl.BlockSpec((1, TN, C2), lambda b, t: (b, t, 0)),
            pl.BlockSpec((1, TN, C1), lambda b, t: (b, t, 0)),
            _full_spec((CO1, IN_CH)),
        ],
        out_specs=[_row_spec(), _full_spec((2, CO1))],
        out_shape=[
            jax.ShapeDtypeStruct((B, N1, CO1), jnp.bfloat16),
            jax.ShapeDtypeStruct((2, CO1), jnp.float32),
        ],
    )(recv, features_1, W1)


def _mlp2_call(h1, st1, gamma1, beta1, W2):
    return pl.pallas_call(
        _mlp2_body,
        grid=(B, N1 // TN),
        in_specs=[
            _row_spec(),
            _full_spec((2, CO1)),
            _full_spec((1, CO1)),
            _full_spec((1, CO1)),
            _full_spec((CO2, CO1)),
        ],
        out_specs=[_row_spec(), _full_spec((2, CO2))],
        out_shape=[
            jax.ShapeDtypeStruct((B, N1, CO2), jnp.bfloat16),
            jax.ShapeDtypeStruct((2, CO2), jnp.float32),
        ],
    )(h1, st1, gamma1.reshape(1, CO1), beta1.reshape(1, CO1), W2)


def _bn3_call(h2, st2, gamma2, beta2):
    return pl.pallas_call(
        _bn_relu_body,
        grid=(B, N1 // TN),
        in_specs=[
            _row_spec(),
            _full_spec((2, CO2)),
            _full_spec((1, CO2)),
            _full_spec((1, CO2)),
        ],
        out_specs=_row_spec(),
        out_shape=jax.ShapeDtypeStruct((B, N1, CO2), jnp.float32),
    )(h2, st2, gamma2.reshape(1, CO2), beta2.reshape(1, CO2))


def kernel(points_1, features_1, points_2, features_2,
           W1, gamma1, beta1, W2, gamma2, beta2):
    idx, w = _knn(points_1, points_2)
    recv = _sc_gather(features_2.reshape(B * N2, C2),
                      idx.reshape(Q * 3), w.reshape(Q * 3))
    h1, st1 = _mlp1_call(recv.reshape(B, N1, C2), features_1, W1)
    h2, st2 = _mlp2_call(h1, st1, gamma1, beta1, W2)
    return _bn3_call(h2, st2, gamma2, beta2)


# speedup vs baseline: 18.5649x; 1.0811x over previous
import functools

import jax
import jax.numpy as jnp
from jax import lax
from jax.experimental import pallas as pl
from jax.experimental.pallas import tpu as pltpu
from jax.experimental.pallas import tpu_sc as plsc

B, N1, N2, C1, C2 = 8, 4096, 1024, 128, 256
CO1, CO2 = 256, 256
IN_CH = C1 + C2
TN = 2048
Q = B * N1
NW = 32
QPW = Q // NW
CH = 32
NCH = QPW // CH


def _knn_body(p1_ref, p2_ref, idx_ref, w_ref):
    b = pl.program_id(0)
    p1 = p1_ref[0]
    p2 = p2_ref[0]
    dot = lax.dot_general(p1, p2, (((1,), (1,)), ((), ())),
                          preferred_element_type=jnp.float32)
    p1s = jnp.sum(p1 * p1, axis=1, keepdims=True)
    p2s = jnp.sum(p2 * p2, axis=1)[None, :]
    d2 = p1s + p2s - 2.0 * dot
    iota = lax.broadcasted_iota(jnp.int32, (TN, N2), 1)
    cur = d2
    vals, idxs = [], []
    for k in range(3):
        mv = jnp.min(cur, axis=1, keepdims=True)
        eq = cur == mv
        mi = jnp.min(jnp.where(eq, iota, N2), axis=1, keepdims=True)
        vals.append(mv)
        idxs.append(mi)
        if k < 2:
            cur = jnp.where(eq, jnp.float32(jnp.inf), cur)
    dist = jnp.concatenate(vals, axis=1)
    ind = jnp.concatenate(idxs, axis=1)
    rec = 1.0 / (dist + 1e-8)
    w_ref[0] = rec / jnp.sum(rec, axis=1, keepdims=True)
    idx_ref[0] = ind + b * N2


def _knn(points_1, points_2):
    return pl.pallas_call(
        _knn_body,
        grid=(B, N1 // TN),
        in_specs=[
            pl.BlockSpec((1, TN, 3), lambda b, t: (b, t, 0)),
            pl.BlockSpec((1, N2, 3), lambda b, t: (b, 0, 0)),
        ],
        out_specs=[
            pl.BlockSpec((1, TN, 3), lambda b, t: (b, t, 0)),
            pl.BlockSpec((1, TN, 3), lambda b, t: (b, t, 0)),
        ],
        out_shape=[
            jax.ShapeDtypeStruct((B, N1, 3), jnp.int32),
            jax.ShapeDtypeStruct((B, N1, 3), jnp.float32),
        ],
    )(points_1, points_2)


@functools.lru_cache(maxsize=1)
def _make_gather_combine():
    mesh = plsc.VectorSubcoreMesh(core_axis_name="c", subcore_axis_name="s")

    @functools.partial(
        pl.kernel,
        mesh=mesh,
        out_type=jax.ShapeDtypeStruct((Q, C2), jnp.float32),
        scratch_types=[
            pltpu.VMEM((3 * QPW,), jnp.int32),
            pltpu.VMEM((3 * QPW + 16,), jnp.float32),
            pltpu.VMEM((2, 3 * CH, C2), jnp.float32),
            pltpu.VMEM((2, CH, C2), jnp.float32),
            pltpu.SemaphoreType.DMA,
            pltpu.SemaphoreType.DMA,
        ],
    )
    def gather_combine(f2_hbm, idx_hbm, w_hbm, out_hbm,
                       idx_v, w_v, rows_v, out_v, gsem, osem):
        wid = lax.axis_index("s") * 2 + lax.axis_index("c")
        qbase = wid * QPW
        pltpu.sync_copy(idx_hbm.at[pl.ds(3 * qbase, 3 * QPW)], idx_v)
        pltpu.sync_copy(w_hbm.at[pl.ds(3 * qbase, 3 * QPW)],
                        w_v.at[pl.ds(0, 3 * QPW)])

        def start_gather(ci, slot):
            pltpu.async_copy(f2_hbm.at[idx_v.at[pl.ds(3 * CH * ci, 3 * CH)]],
                             rows_v.at[slot], gsem)

        def combine(ci, slot):
            def per_q(qi, c):
                wvec = w_v[pl.ds(3 * CH * ci + 3 * qi, 16)]
                w0 = wvec[0]
                w1 = wvec[1]
                w2 = wvec[2]
                for j in range(C2 // 16):
                    sl = pl.ds(j * 16, 16)
                    out_v[slot, qi, sl] = (w0 * rows_v[slot, 3 * qi, sl]
                                           + w1 * rows_v[slot, 3 * qi + 1, sl]
                                           + w2 * rows_v[slot, 3 * qi + 2, sl])
                return c

            lax.fori_loop(0, CH, per_q, 0)

        def wait_gather(slot):
            pltpu.make_async_copy(f2_hbm.at[pl.ds(0, 3 * CH)],
                                  rows_v.at[slot], gsem).wait()

        def wait_out(slot):
            pltpu.make_async_copy(out_v.at[slot],
                                  out_hbm.at[pl.ds(qbase, CH)], osem).wait()

        start_gather(0, 0)

        def step(i2, carry):
            for b in range(2):
                ci = 2 * i2 + b
                slot = b
                nxt = ci + 1

                @pl.when(nxt < NCH)
                def _pre():
                    start_gather(nxt, 1 - slot)

                wait_gather(slot)

                @pl.when(ci >= 2)
                def _drain():
                    wait_out(slot)

                combine(ci, slot)
                pltpu.async_copy(out_v.at[slot],
                                 out_hbm.at[pl.ds(qbase + ci * CH, CH)], osem)
            return carry

        lax.fori_loop(0, NCH // 2, step, 0)
        wait_out(0)
        wait_out(1)

    return gather_combine


def _sc_gather(f2flat, idxf, wf):
    return _make_gather_combine()(f2flat, idxf, wf)


def _mlp1_body(recv_ref, f1_ref, w1_ref, h_ref, st_ref):
    x = jnp.concatenate([recv_ref[0], f1_ref[0]], axis=1)
    h = lax.dot_general(x, w1_ref[...], (((1,), (1,)), ((), ())),
                        preferred_element_type=jnp.float32)

    @pl.when((pl.program_id(0) == 0) & (pl.program_id(1) == 0))
    def _init():
        st_ref[...] = jnp.zeros_like(st_ref)

    h_ref[0] = h.astype(jnp.bfloat16)
    st_ref[0:1, :] += jnp.sum(h, axis=0, keepdims=True)
    st_ref[1:2, :] += jnp.sum(h * h, axis=0, keepdims=True)


def _mlp2_body(h1_ref, st1_ref, g1_ref, b1_ref, w2_ref, h2_ref, st2_ref):
    n = jnp.float32(B * N1)
    mean = st1_ref[0:1, :] / n
    var = st1_ref[1:2, :] / n - mean * mean
    scale = g1_ref[...] * lax.rsqrt(var + 1e-5)
    shift = b1_ref[...] - mean * scale
    x = jnp.maximum(h1_ref[0].astype(jnp.float32) * scale + shift, 0.0)
    h2 = lax.dot_general(x, w2_ref[...], (((1,), (1,)), ((), ())),
                         preferred_element_type=jnp.float32)

    @pl.when((pl.program_id(0) == 0) & (pl.program_id(1) == 0))
    def _init():
        st2_ref[...] = jnp.zeros_like(st2_ref)

    h2_ref[0] = h2.astype(jnp.bfloat16)
    st2_ref[0:1, :] += jnp.sum(h2, axis=0, keepdims=True)
    st2_ref[1:2, :] += jnp.sum(h2 * h2, axis=0, keepdims=True)


def _bn_relu_body(h_ref, st_ref, g_ref, b_ref, o_ref):
    n = jnp.float32(B * N1)
    mean = st_ref[0:1, :] / n
    var = st_ref[1:2, :] / n - mean * mean
    scale = g_ref[...] * lax.rsqrt(var + 1e-5)
    shift = b_ref[...] - mean * scale
    o_ref[0] = jnp.maximum(h_ref[0].astype(jnp.float32) * scale + shift, 0.0)


def _row_spec():
    return pl.BlockSpec((1, TN, CO1), lambda b, t: (b, t, 0))


def _full_spec(shape):
    return pl.BlockSpec(shape, lambda b, t: tuple(0 for _ in shape))


def _mlp1_call(recv, features_1, W1):
    return pl.pallas_call(
        _mlp1_body,
        grid=(B, N1 // TN),
        in_specs=[
            pl.BlockSpec((1, TN, C2), lambda b, t: (b, t, 0)),
            pl.BlockSpec((1, TN, C1), lambda b, t: (b, t, 0)),
            _full_spec((CO1, IN_CH)),
        ],
        out_specs=[_row_spec(), _full_spec((2, CO1))],
        out_shape=[
            jax.ShapeDtypeStruct((B, N1, CO1), jnp.bfloat16),
            jax.ShapeDtypeStruct((2, CO1), jnp.float32),
        ],
    )(recv, features_1, W1)


def _mlp2_call(h1, st1, gamma1, beta1, W2):
    return pl.pallas_call(
        _mlp2_body,
        grid=(B, N1 // TN),
        in_specs=[
            _row_spec(),
            _full_spec((2, CO1)),
            _full_spec((1, CO1)),
            _full_spec((1, CO1)),
            _full_spec((CO2, CO1)),
        ],
        out_specs=[_row_spec(), _full_spec((2, CO2))],
        out_shape=[
            jax.ShapeDtypeStruct((B, N1, CO2), jnp.bfloat16),
            jax.ShapeDtypeStruct((2, CO2), jnp.float32),
        ],
    )(h1, st1, gamma1.reshape(1, CO1), beta1.reshape(1, CO1), W2)


def _bn3_call(h2, st2, gamma2, beta2):
    return pl.pallas_call(
        _bn_relu_body,
        grid=(B, N1 // TN),
        in_specs=[
            _row_spec(),
            _full_spec((2, CO2)),
            _full_spec((1, CO2)),
            _full_spec((1, CO2)),
        ],
        out_specs=_row_spec(),
        out_shape=jax.ShapeDtypeStruct((B, N1, CO2), jnp.float32),
    )(h2, st2, gamma2.reshape(1, CO2), beta2.reshape(1, CO2))


def kernel(points_1, features_1, points_2, features_2,
           W1, gamma1, beta1, W2, gamma2, beta2):
    idx, w = _knn(points_1, points_2)
    recv = _sc_gather(features_2.reshape(B * N2, C2),
                      idx.reshape(Q * 3), w.reshape(Q * 3))
    h1, st1 = _mlp1_call(recv.reshape(B, N1, C2), features_1, W1)
    h2, st2 = _mlp2_call(h1, st1, gamma1, beta1, W2)
    return _bn3_call(h2, st2, gamma2, beta2)
